# baseline (speedup 1.0000x reference)
import functools

import jax
import jax.numpy as jnp
from jax.experimental import pallas as pl
from jax.experimental.pallas import tpu as pltpu
from jax.experimental.pallas import tpu_sc as plsc

N_POINTS = 8192
OUT_CHANNELS = 128
MAX_NBRS = 64

_NC = 2
_NS = 16
_NW = _NC * _NS
_BIG = 2 ** 30


def _fps_body(n_samples, n_points, xs_ref, ys_ref, zs_ref, sel_ref):
    xs = xs_ref[...]
    ys = ys_ref[...]
    zs = zs_ref[...]
    rows, lanes = xs.shape
    flat_iota = (jax.lax.broadcasted_iota(jnp.int32, (rows, lanes), 0) * lanes
                 + jax.lax.broadcasted_iota(jnp.int32, (rows, lanes), 1))
    srows = sel_ref.shape[0]
    sel_iota = (jax.lax.broadcasted_iota(jnp.int32, (srows, lanes), 0) * lanes
                + jax.lax.broadcasted_iota(jnp.int32, (srows, lanes), 1))

    zero = jnp.float32(0.0)

    def body(i, state):
        sel_vec, dists, lx, ly, lz = state
        dx = xs - lx
        dy = ys - ly
        dz = zs - lz
        d = dx * dx + dy * dy + dz * dz
        dists = jnp.minimum(dists, d)
        m = jnp.max(dists)
        idx = jnp.min(jnp.where(dists == m, flat_iota, n_points))
        sel_vec = jnp.where(sel_iota == i, idx, sel_vec)
        pick = flat_iota == idx
        lx = jnp.sum(jnp.where(pick, xs, zero))
        ly = jnp.sum(jnp.where(pick, ys, zero))
        lz = jnp.sum(jnp.where(pick, zs, zero))
        return (sel_vec, dists, lx, ly, lz)

    sel0 = jnp.zeros((srows, lanes), dtype=jnp.int32)
    d0 = jnp.full((rows, lanes), jnp.inf, dtype=jnp.float32)
    pick0 = flat_iota == 0
    lx0 = jnp.sum(jnp.where(pick0, xs, zero))
    ly0 = jnp.sum(jnp.where(pick0, ys, zero))
    lz0 = jnp.sum(jnp.where(pick0, zs, zero))
    sel, _, _, _, _ = jax.lax.fori_loop(
        1, n_samples, body, (sel0, d0, lx0, ly0, lz0))
    sel_ref[...] = sel


def _fps(pos, n_samples):
    n = pos.shape[0]
    xs = pos[:, 0].reshape(n // 128, 128)
    ys = pos[:, 1].reshape(n // 128, 128)
    zs = pos[:, 2].reshape(n // 128, 128)
    sel = pl.pallas_call(
        functools.partial(_fps_body, n_samples, n),
        out_shape=jax.ShapeDtypeStruct((n_samples // 128, 128), jnp.int32),
    )(xs, ys, zs)
    return sel.reshape(n_samples)


def _d2_body(dst_ref, srcT_ref, out_ref):
    q = dst_ref[...]
    sT = srcT_ref[...]
    qs2 = jnp.sum(q * q, axis=1, keepdims=True)
    ss2 = jnp.sum(sT * sT, axis=0, keepdims=True)
    cross = jnp.dot(q, sT, preferred_element_type=jnp.float32)
    out_ref[...] = qs2 + ss2 - 2.0 * cross


def _d2(pos_dst8, pos_src8T):
    qn = pos_dst8.shape[0]
    n = pos_src8T.shape[1]
    bd = 256
    return pl.pallas_call(
        _d2_body,
        grid=(qn // bd,),
        in_specs=[
            pl.BlockSpec((bd, 8), lambda i: (i, 0)),
            pl.BlockSpec((8, n), lambda i: (0, 0)),
        ],
        out_specs=pl.BlockSpec((bd, n), lambda i: (i, 0)),
        out_shape=jax.ShapeDtypeStruct((qn, n), jnp.float32),
    )(pos_dst8, pos_src8T)


def _scan_body(rows_pw, n, ch, r2, d2_hbm, tab_hbm, gath_hbm,
               chunk_v, idx_v, gath_v, sem):
    cid = jax.lax.axis_index("c")
    sid = jax.lax.axis_index("s")
    wid = sid * _NC + cid
    iota16 = jax.lax.iota(jnp.int32, 16)
    nchunks = n // ch
    nvr = ch // 16

    def row_body(rl, carry):
        q = wid * rows_pw + rl
        pltpu.async_copy(d2_hbm.at[q, pl.ds(0, ch)], chunk_v.at[0], sem)

        def cond(st):
            i, cnt = st
            return jnp.logical_and(i < nchunks, cnt < MAX_NBRS)

        def cbody(st):
            i, cnt = st
            b = jax.lax.rem(i, 2)
            pltpu.make_async_copy(
                d2_hbm.at[q, pl.ds(0, ch)], chunk_v.at[b], sem).wait()

            @pl.when(i + 1 < nchunks)
            def _():
                pltpu.async_copy(
                    d2_hbm.at[q, pl.ds((i + 1) * ch, ch)],
                    chunk_v.at[jax.lax.rem(i + 1, 2)], sem)

            base = i * ch

            def vcond(st2):
                v, cnt2 = st2
                return jnp.logical_and(v < nvr, cnt2 < MAX_NBRS)

            def vbody(st2):
                v, cnt2 = st2
                d = chunk_v[b, pl.ds(v * 16, 16)]
                m = d <= r2

                def busy(c):
                    csum = plsc.cumsum(m.astype(jnp.int32))
                    slot = c + csum - 1
                    glob = iota16 + (base + v * 16)
                    msel = jnp.logical_and(m, slot < MAX_NBRS)
                    plsc.store_scatter(idx_v, [slot], glob, mask=msel)
                    return c + csum[15]

                cnt2 = jax.lax.cond(jnp.any(m), busy, lambda c: c, cnt2)
                return (v + 1, cnt2)

            _, cnt = jax.lax.while_loop(vcond, vbody, (jnp.int32(0), cnt))
            return (i + 1, cnt)

        i_exit, cnt = jax.lax.while_loop(
            cond, cbody, (jnp.int32(0), jnp.int32(0)))

        @pl.when(i_exit < nchunks)
        def _():
            pltpu.make_async_copy(
                d2_hbm.at[q, pl.ds(0, ch)], chunk_v.at[0], sem).wait()

        first = idx_v[pl.ds(0, 16)][0]
        for k in range(MAX_NBRS // 16):
            sp = iota16 + k * 16
            cur = idx_v[pl.ds(k * 16, 16)]
            idx_v[pl.ds(k * 16, 16)] = jnp.where(sp < cnt, cur, first)

        pltpu.async_copy(tab_hbm.at[idx_v], gath_v, sem).wait()
        pltpu.sync_copy(gath_v, gath_hbm.at[pl.ds(q * MAX_NBRS, MAX_NBRS)])
        return carry

    jax.lax.fori_loop(0, rows_pw, row_body, jnp.int32(0))


def _scan_gather(d2, table, r2, ch):
    qn, n = d2.shape
    p = table.shape[1]
    rows_pw = qn // _NW
    mesh = plsc.VectorSubcoreMesh(
        core_axis_name="c", subcore_axis_name="s",
        num_cores=_NC, num_subcores=_NS)
    return pl.kernel(
        functools.partial(_scan_body, rows_pw, n, ch, r2),
        out_type=jax.ShapeDtypeStruct((qn * MAX_NBRS, p), jnp.float32),
        mesh=mesh,
        scratch_types=[
            pltpu.VMEM((2, ch), jnp.float32),
            pltpu.VMEM((MAX_NBRS,), jnp.int32),
            pltpu.VMEM((MAX_NBRS, p), jnp.float32),
            pltpu.SemaphoreType.DMA,
        ],
        compiler_params=pltpu.CompilerParams(needs_layout_passes=False),
    )(d2, table)


def _conv_body(bq, hdim, gath_ref, posd_ref, w1_ref, w1p_ref, b1_ref,
               w2_ref, b2_ref, out_ref):
    g = gath_ref[...]
    e = jnp.dot(g, w1_ref[...],
                preferred_element_type=jnp.float32) + b1_ref[...]
    wq = jnp.dot(posd_ref[...], w1p_ref[...],
                 preferred_element_type=jnp.float32)
    e3 = e.reshape(bq, MAX_NBRS, hdim)
    h3 = jnp.maximum(e3 - wq[:, None, :], 0.0)
    msg = jnp.dot(h3.reshape(bq * MAX_NBRS, hdim), w2_ref[...],
                  preferred_element_type=jnp.float32)
    m3 = msg.reshape(bq, MAX_NBRS, hdim)
    out_ref[...] = jnp.max(m3, axis=1) + b2_ref[...]


def _conv(gath, pos_dst8, w1pad, w1p8, b1, w2, b2):
    qn = pos_dst8.shape[0]
    p = gath.shape[1]
    hdim = w2.shape[0]
    bq = 128
    return pl.pallas_call(
        functools.partial(_conv_body, bq, hdim),
        grid=(qn // bq,),
        in_specs=[
            pl.BlockSpec((bq * MAX_NBRS, p), lambda i: (i, 0)),
            pl.BlockSpec((bq, 8), lambda i: (i, 0)),
            pl.BlockSpec(w1pad.shape, lambda i: (0, 0)),
            pl.BlockSpec(w1p8.shape, lambda i: (0, 0)),
            pl.BlockSpec((1, hdim), lambda i: (0, 0)),
            pl.BlockSpec(w2.shape, lambda i: (0, 0)),
            pl.BlockSpec((1, hdim), lambda i: (0, 0)),
        ],
        out_specs=pl.BlockSpec((bq, hdim), lambda i: (i, 0)),
        out_shape=jax.ShapeDtypeStruct((qn, hdim), jnp.float32),
    )(gath, pos_dst8, w1pad, w1p8, b1.reshape(1, hdim), w2,
      b2.reshape(1, hdim))


def _pad_cols(a, width):
    return jnp.pad(a, ((0, 0), (0, width - a.shape[1])))


def _stage(pos_src, pos_dst, x_src, w1, b1, w2, b2, r, ch):
    n = pos_src.shape[0]
    f = x_src.shape[1]
    hdim = w2.shape[0]
    p = 128
    table = _pad_cols(jnp.concatenate([x_src, pos_src], axis=1), p)
    pos_dst8 = _pad_cols(pos_dst, 8)
    pos_src8t = _pad_cols(pos_src, 8).T
    d2 = _d2(pos_dst8, pos_src8t)
    gath = _scan_gather(d2, table, jnp.float32(r * r), ch)
    w1pad = jnp.pad(w1, ((0, p - (f + 3)), (0, 0)))
    w1p8 = jnp.pad(w1[f:f + 3], ((0, 5), (0, 0)))
    return _conv(gath, pos_dst8, w1pad, w1p8, b1, w2, b2)


def kernel(pos, W11, b11, W12, b12, W21, b21, W22, b22, W31, b31, W32, b32, batch):
    n1 = N_POINTS // 2
    idx1 = _fps(pos, n1)
    pos1 = jnp.take(pos, idx1, axis=0)
    x1 = _stage(pos, pos1, pos, W11, b11, W12, b12, 0.2, 4096)

    n2 = n1 // 4
    idx2 = _fps(pos1, n2)
    pos2 = jnp.take(pos1, idx2, axis=0)
    x2 = _stage(pos1, pos2, x1, W21, b21, W22, b22, 0.4, 512)

    x3 = _stage(pos2, pos2, x2, W31, b31, W32, b32, 1.0, 512)

    batch3 = jnp.take(jnp.take(batch, idx1), idx2)
    return (x3, pos2, batch3)


# speedup vs baseline: 1.1057x; 1.1057x over previous
import functools

import jax
import jax.numpy as jnp
from jax.experimental import pallas as pl
from jax.experimental.pallas import tpu as pltpu
from jax.experimental.pallas import tpu_sc as plsc

N_POINTS = 8192
OUT_CHANNELS = 128
MAX_NBRS = 64

_NC = 2
_NS = 16
_NW = _NC * _NS
_BIG = 2 ** 30


def _fps_body(n_samples, n_points, xs_ref, ys_ref, zs_ref, sel_ref):
    xs = xs_ref[...]
    ys = ys_ref[...]
    zs = zs_ref[...]
    rows, lanes = xs.shape
    flat_iota = (jax.lax.broadcasted_iota(jnp.int32, (rows, lanes), 0) * lanes
                 + jax.lax.broadcasted_iota(jnp.int32, (rows, lanes), 1))
    srows = sel_ref.shape[0]
    sel_iota = (jax.lax.broadcasted_iota(jnp.int32, (srows, lanes), 0) * lanes
                + jax.lax.broadcasted_iota(jnp.int32, (srows, lanes), 1))

    zero = jnp.float32(0.0)

    def body(i, state):
        sel_vec, dists, lx, ly, lz = state
        dx = xs - lx
        dy = ys - ly
        dz = zs - lz
        d = dx * dx + dy * dy + dz * dz
        dists = jnp.minimum(dists, d)
        m = jnp.max(dists)
        idx = jnp.min(jnp.where(dists == m, flat_iota, n_points))
        sel_vec = jnp.where(sel_iota == i, idx, sel_vec)
        pick = flat_iota == idx
        lx = jnp.sum(jnp.where(pick, xs, zero))
        ly = jnp.sum(jnp.where(pick, ys, zero))
        lz = jnp.sum(jnp.where(pick, zs, zero))
        return (sel_vec, dists, lx, ly, lz)

    sel0 = jnp.zeros((srows, lanes), dtype=jnp.int32)
    d0 = jnp.full((rows, lanes), jnp.inf, dtype=jnp.float32)
    pick0 = flat_iota == 0
    lx0 = jnp.sum(jnp.where(pick0, xs, zero))
    ly0 = jnp.sum(jnp.where(pick0, ys, zero))
    lz0 = jnp.sum(jnp.where(pick0, zs, zero))
    sel, _, _, _, _ = jax.lax.fori_loop(
        1, n_samples, body, (sel0, d0, lx0, ly0, lz0))
    sel_ref[...] = sel


def _fps(pos, n_samples):
    n = pos.shape[0]
    xs = pos[:, 0].reshape(n // 128, 128)
    ys = pos[:, 1].reshape(n // 128, 128)
    zs = pos[:, 2].reshape(n // 128, 128)
    sel = pl.pallas_call(
        functools.partial(_fps_body, n_samples, n),
        out_shape=jax.ShapeDtypeStruct((n_samples // 128, 128), jnp.int32),
    )(xs, ys, zs)
    return sel.reshape(n_samples)


def _d2_body(dst_ref, srcT_ref, out_ref):
    q = dst_ref[...]
    sT = srcT_ref[...]
    qs2 = jnp.sum(q * q, axis=1, keepdims=True)
    ss2 = jnp.sum(sT * sT, axis=0, keepdims=True)
    cross = jnp.dot(q, sT, preferred_element_type=jnp.float32)
    out_ref[...] = qs2 + ss2 - 2.0 * cross


def _d2(pos_dst8, pos_src8T):
    qn = pos_dst8.shape[0]
    n = pos_src8T.shape[1]
    bd = 256
    return pl.pallas_call(
        _d2_body,
        grid=(qn // bd,),
        in_specs=[
            pl.BlockSpec((bd, 8), lambda i: (i, 0)),
            pl.BlockSpec((8, n), lambda i: (0, 0)),
        ],
        out_specs=pl.BlockSpec((bd, n), lambda i: (i, 0)),
        out_shape=jax.ShapeDtypeStruct((qn, n), jnp.float32),
    )(pos_dst8, pos_src8T)


def _scan_body(rows_pw, n, ch, r2, d2_hbm, tab_hbm, gath_hbm,
               chunk_v, idx_v, gath_v, sem):
    cid = jax.lax.axis_index("c")
    sid = jax.lax.axis_index("s")
    wid = sid * _NC + cid
    iota16 = jax.lax.iota(jnp.int32, 16)
    nchunks = n // ch
    nvr = ch // 16

    def row_body(rl, carry):
        q = wid * rows_pw + rl
        pltpu.async_copy(d2_hbm.at[q, pl.ds(0, ch)], chunk_v.at[0], sem)

        def cond(st):
            i, cnt = st
            return jnp.logical_and(i < nchunks, cnt < MAX_NBRS)

        def cbody(st):
            i, cnt = st
            b = jax.lax.rem(i, 2)
            pltpu.make_async_copy(
                d2_hbm.at[q, pl.ds(0, ch)], chunk_v.at[b], sem).wait()

            @pl.when(i + 1 < nchunks)
            def _():
                pltpu.async_copy(
                    d2_hbm.at[q, pl.ds((i + 1) * ch, ch)],
                    chunk_v.at[jax.lax.rem(i + 1, 2)], sem)

            base = i * ch

            def vbody(v, cnt2):
                d = chunk_v[b, pl.ds(v * 16, 16)]
                m = d <= r2
                nv = plsc.all_reduce_population_count(m)[0]

                @pl.when(nv > 0)
                def _():
                    csum = plsc.cumsum(m.astype(jnp.int32))
                    slot = cnt2 + csum - 1
                    glob = iota16 + (base + v * 16)
                    msel = jnp.logical_and(m, slot < MAX_NBRS)
                    plsc.store_scatter(idx_v, [slot], glob, mask=msel)

                return cnt2 + nv

            cnt = jax.lax.fori_loop(0, nvr, vbody, cnt)
            return (i + 1, cnt)

        i_exit, cnt = jax.lax.while_loop(
            cond, cbody, (jnp.int32(0), jnp.int32(0)))

        @pl.when(i_exit < nchunks)
        def _():
            pltpu.make_async_copy(
                d2_hbm.at[q, pl.ds(0, ch)], chunk_v.at[0], sem).wait()

        first = idx_v[pl.ds(0, 16)][0]
        for k in range(MAX_NBRS // 16):
            sp = iota16 + k * 16
            cur = idx_v[pl.ds(k * 16, 16)]
            idx_v[pl.ds(k * 16, 16)] = jnp.where(sp < cnt, cur, first)

        pltpu.async_copy(tab_hbm.at[idx_v], gath_v, sem).wait()
        pltpu.sync_copy(gath_v, gath_hbm.at[pl.ds(q * MAX_NBRS, MAX_NBRS)])
        return carry

    jax.lax.fori_loop(0, rows_pw, row_body, jnp.int32(0))


def _scan_gather(d2, table, r2, ch):
    qn, n = d2.shape
    p = table.shape[1]
    rows_pw = qn // _NW
    mesh = plsc.VectorSubcoreMesh(
        core_axis_name="c", subcore_axis_name="s",
        num_cores=_NC, num_subcores=_NS)
    return pl.kernel(
        functools.partial(_scan_body, rows_pw, n, ch, r2),
        out_type=jax.ShapeDtypeStruct((qn * MAX_NBRS, p), jnp.float32),
        mesh=mesh,
        scratch_types=[
            pltpu.VMEM((2, ch), jnp.float32),
            pltpu.VMEM((MAX_NBRS,), jnp.int32),
            pltpu.VMEM((MAX_NBRS, p), jnp.float32),
            pltpu.SemaphoreType.DMA,
        ],
        compiler_params=pltpu.CompilerParams(needs_layout_passes=False),
    )(d2, table)


def _conv_body(bq, hdim, gath_ref, posd_ref, w1_ref, w1p_ref, b1_ref,
               w2_ref, b2_ref, out_ref):
    g = gath_ref[...]
    e = jnp.dot(g, w1_ref[...],
                preferred_element_type=jnp.float32) + b1_ref[...]
    wq = jnp.dot(posd_ref[...], w1p_ref[...],
                 preferred_element_type=jnp.float32)
    e3 = e.reshape(bq, MAX_NBRS, hdim)
    h3 = jnp.maximum(e3 - wq[:, None, :], 0.0)
    msg = jnp.dot(h3.reshape(bq * MAX_NBRS, hdim), w2_ref[...],
                  preferred_element_type=jnp.float32)
    m3 = msg.reshape(bq, MAX_NBRS, hdim)
    out_ref[...] = jnp.max(m3, axis=1) + b2_ref[...]


def _conv(gath, pos_dst8, w1pad, w1p8, b1, w2, b2):
    qn = pos_dst8.shape[0]
    p = gath.shape[1]
    hdim = w2.shape[0]
    bq = 128
    return pl.pallas_call(
        functools.partial(_conv_body, bq, hdim),
        grid=(qn // bq,),
        in_specs=[
            pl.BlockSpec((bq * MAX_NBRS, p), lambda i: (i, 0)),
            pl.BlockSpec((bq, 8), lambda i: (i, 0)),
            pl.BlockSpec(w1pad.shape, lambda i: (0, 0)),
            pl.BlockSpec(w1p8.shape, lambda i: (0, 0)),
            pl.BlockSpec((1, hdim), lambda i: (0, 0)),
            pl.BlockSpec(w2.shape, lambda i: (0, 0)),
            pl.BlockSpec((1, hdim), lambda i: (0, 0)),
        ],
        out_specs=pl.BlockSpec((bq, hdim), lambda i: (i, 0)),
        out_shape=jax.ShapeDtypeStruct((qn, hdim), jnp.float32),
    )(gath, pos_dst8, w1pad, w1p8, b1.reshape(1, hdim), w2,
      b2.reshape(1, hdim))


def _pad_cols(a, width):
    return jnp.pad(a, ((0, 0), (0, width - a.shape[1])))


def _stage(pos_src, pos_dst, x_src, w1, b1, w2, b2, r, ch):
    n = pos_src.shape[0]
    f = x_src.shape[1]
    hdim = w2.shape[0]
    p = 128
    table = _pad_cols(jnp.concatenate([x_src, pos_src], axis=1), p)
    pos_dst8 = _pad_cols(pos_dst, 8)
    pos_src8t = _pad_cols(pos_src, 8).T
    d2 = _d2(pos_dst8, pos_src8t)
    gath = _scan_gather(d2, table, jnp.float32(r * r), ch)
    w1pad = jnp.pad(w1, ((0, p - (f + 3)), (0, 0)))
    w1p8 = jnp.pad(w1[f:f + 3], ((0, 5), (0, 0)))
    return _conv(gath, pos_dst8, w1pad, w1p8, b1, w2, b2)


def kernel(pos, W11, b11, W12, b12, W21, b21, W22, b22, W31, b31, W32, b32, batch):
    n1 = N_POINTS // 2
    idx1 = _fps(pos, n1)
    pos1 = jnp.take(pos, idx1, axis=0)
    x1 = _stage(pos, pos1, pos, W11, b11, W12, b12, 0.2, 1024)

    n2 = n1 // 4
    idx2 = _fps(pos1, n2)
    pos2 = jnp.take(pos1, idx2, axis=0)
    x2 = _stage(pos1, pos2, x1, W21, b21, W22, b22, 0.4, 512)

    x3 = _stage(pos2, pos2, x2, W31, b31, W32, b32, 1.0, 512)

    batch3 = jnp.take(jnp.take(batch, idx1), idx2)
    return (x3, pos2, batch3)


# speedup vs baseline: 1.2454x; 1.1263x over previous
import functools

import jax
import jax.numpy as jnp
from jax.experimental import pallas as pl
from jax.experimental.pallas import tpu as pltpu
from jax.experimental.pallas import tpu_sc as plsc

N_POINTS = 8192
OUT_CHANNELS = 128
MAX_NBRS = 64

_NC = 2
_NS = 16
_NW = _NC * _NS
_BIG = 2 ** 30


def _fps_body(n_samples, n_points, xs_ref, ys_ref, zs_ref,
              xc_ref, yc_ref, zc_ref, sel_ref):
    xs = xs_ref[...]
    ys = ys_ref[...]
    zs = zs_ref[...]
    rows, lanes = xs.shape
    flat_iota = (jax.lax.broadcasted_iota(jnp.int32, (rows, lanes), 0) * lanes
                 + jax.lax.broadcasted_iota(jnp.int32, (rows, lanes), 1))

    srows = sel_ref.shape[0]
    sel_iota = (jax.lax.broadcasted_iota(jnp.int32, (srows, lanes), 0) * lanes
                + jax.lax.broadcasted_iota(jnp.int32, (srows, lanes), 1))

    def body(i, state):
        sel_vec, dists, lx, ly, lz = state
        dx = xs - lx
        dy = ys - ly
        dz = zs - lz
        d = dx * dx + dy * dy + dz * dz
        dists = jnp.minimum(dists, d)
        m = jnp.max(dists)
        idx = jnp.min(jnp.where(dists == m, flat_iota, n_points))
        sel_vec = jnp.where(sel_iota == i, idx, sel_vec)
        lx = xc_ref[idx, 0]
        ly = yc_ref[idx, 0]
        lz = zc_ref[idx, 0]
        return (sel_vec, dists, lx, ly, lz)

    sel0 = jnp.zeros((srows, lanes), dtype=jnp.int32)
    d0 = jnp.full((rows, lanes), jnp.inf, dtype=jnp.float32)
    sel, _, _, _, _ = jax.lax.fori_loop(
        1, n_samples, body,
        (sel0, d0, xc_ref[0, 0], yc_ref[0, 0], zc_ref[0, 0]))
    sel_ref[...] = sel


def _fps(pos, n_samples):
    n = pos.shape[0]
    xs = pos[:, 0].reshape(n // 128, 128)
    ys = pos[:, 1].reshape(n // 128, 128)
    zs = pos[:, 2].reshape(n // 128, 128)
    xc = pos[:, 0].reshape(n, 1)
    yc = pos[:, 1].reshape(n, 1)
    zc = pos[:, 2].reshape(n, 1)
    sel = pl.pallas_call(
        functools.partial(_fps_body, n_samples, n),
        out_shape=jax.ShapeDtypeStruct((n_samples // 128, 128), jnp.int32),
    )(xs, ys, zs, xc, yc, zc)
    return sel.reshape(n_samples)


def _d2_body(dst_ref, srcT_ref, out_ref):
    q = dst_ref[...]
    sT = srcT_ref[...]
    qs2 = jnp.sum(q * q, axis=1, keepdims=True)
    ss2 = jnp.sum(sT * sT, axis=0, keepdims=True)
    cross = jnp.dot(q, sT, preferred_element_type=jnp.float32)
    out_ref[...] = qs2 + ss2 - 2.0 * cross


def _d2(pos_dst8, pos_src8T):
    qn = pos_dst8.shape[0]
    n = pos_src8T.shape[1]
    bd = 256
    return pl.pallas_call(
        _d2_body,
        grid=(qn // bd,),
        in_specs=[
            pl.BlockSpec((bd, 8), lambda i: (i, 0)),
            pl.BlockSpec((8, n), lambda i: (0, 0)),
        ],
        out_specs=pl.BlockSpec((bd, n), lambda i: (i, 0)),
        out_shape=jax.ShapeDtypeStruct((qn, n), jnp.float32),
    )(pos_dst8, pos_src8T)


def _scan_body(rows_pw, n, ch, r2, d2_hbm, tab_hbm, gath_hbm,
               chunk_v, idx_v, gath_v, sem):
    cid = jax.lax.axis_index("c")
    sid = jax.lax.axis_index("s")
    wid = sid * _NC + cid
    iota16 = jax.lax.iota(jnp.int32, 16)
    nchunks = n // ch
    nvr = ch // 16

    def row_body(rl, carry):
        q = wid * rows_pw + rl
        pltpu.async_copy(d2_hbm.at[q, pl.ds(0, ch)], chunk_v.at[0], sem)

        def cond(st):
            i, cnt = st
            return jnp.logical_and(i < nchunks, cnt < MAX_NBRS)

        def cbody(st):
            i, cnt = st
            b = jax.lax.rem(i, 2)
            pltpu.make_async_copy(
                d2_hbm.at[q, pl.ds(0, ch)], chunk_v.at[b], sem).wait()

            @pl.when(i + 1 < nchunks)
            def _():
                pltpu.async_copy(
                    d2_hbm.at[q, pl.ds((i + 1) * ch, ch)],
                    chunk_v.at[jax.lax.rem(i + 1, 2)], sem)

            base = i * ch

            def vbody(v, cnt2):
                d = chunk_v[b, pl.ds(v * 16, 16)]
                m = d <= r2
                nv = plsc.all_reduce_population_count(m)[0]

                @pl.when(nv > 0)
                def _():
                    csum = plsc.cumsum(m.astype(jnp.int32))
                    slot = cnt2 + csum - 1
                    glob = iota16 + (base + v * 16)
                    msel = jnp.logical_and(m, slot < MAX_NBRS)
                    plsc.store_scatter(idx_v, [slot], glob, mask=msel)

                return cnt2 + nv

            cnt = jax.lax.fori_loop(0, nvr, vbody, cnt)
            return (i + 1, cnt)

        i_exit, cnt = jax.lax.while_loop(
            cond, cbody, (jnp.int32(0), jnp.int32(0)))

        @pl.when(i_exit < nchunks)
        def _():
            pltpu.make_async_copy(
                d2_hbm.at[q, pl.ds(0, ch)], chunk_v.at[0], sem).wait()

        first = idx_v[pl.ds(0, 16)][0]
        for k in range(MAX_NBRS // 16):
            sp = iota16 + k * 16
            cur = idx_v[pl.ds(k * 16, 16)]
            idx_v[pl.ds(k * 16, 16)] = jnp.where(sp < cnt, cur, first)

        pltpu.async_copy(tab_hbm.at[idx_v], gath_v, sem).wait()
        pltpu.sync_copy(gath_v, gath_hbm.at[pl.ds(q * MAX_NBRS, MAX_NBRS)])
        return carry

    jax.lax.fori_loop(0, rows_pw, row_body, jnp.int32(0))


def _scan_gather(d2, table, r2, ch):
    qn, n = d2.shape
    p = table.shape[1]
    rows_pw = qn // _NW
    mesh = plsc.VectorSubcoreMesh(
        core_axis_name="c", subcore_axis_name="s",
        num_cores=_NC, num_subcores=_NS)
    return pl.kernel(
        functools.partial(_scan_body, rows_pw, n, ch, r2),
        out_type=jax.ShapeDtypeStruct((qn * MAX_NBRS, p), jnp.float32),
        mesh=mesh,
        scratch_types=[
            pltpu.VMEM((2, ch), jnp.float32),
            pltpu.VMEM((MAX_NBRS,), jnp.int32),
            pltpu.VMEM((MAX_NBRS, p), jnp.float32),
            pltpu.SemaphoreType.DMA,
        ],
        compiler_params=pltpu.CompilerParams(needs_layout_passes=False),
    )(d2, table)


def _conv_body(bq, hdim, gath_ref, posd_ref, w1_ref, w1p_ref, b1_ref,
               w2_ref, b2_ref, out_ref):
    g = gath_ref[...]
    e = jnp.dot(g, w1_ref[...],
                preferred_element_type=jnp.float32) + b1_ref[...]
    wq = jnp.dot(posd_ref[...], w1p_ref[...],
                 preferred_element_type=jnp.float32)
    e3 = e.reshape(bq, MAX_NBRS, hdim)
    h3 = jnp.maximum(e3 - wq[:, None, :], 0.0)
    msg = jnp.dot(h3.reshape(bq * MAX_NBRS, hdim), w2_ref[...],
                  preferred_element_type=jnp.float32)
    m3 = msg.reshape(bq, MAX_NBRS, hdim)
    out_ref[...] = jnp.max(m3, axis=1) + b2_ref[...]


def _conv(gath, pos_dst8, w1pad, w1p8, b1, w2, b2):
    qn = pos_dst8.shape[0]
    p = gath.shape[1]
    hdim = w2.shape[0]
    bq = 128
    return pl.pallas_call(
        functools.partial(_conv_body, bq, hdim),
        grid=(qn // bq,),
        in_specs=[
            pl.BlockSpec((bq * MAX_NBRS, p), lambda i: (i, 0)),
            pl.BlockSpec((bq, 8), lambda i: (i, 0)),
            pl.BlockSpec(w1pad.shape, lambda i: (0, 0)),
            pl.BlockSpec(w1p8.shape, lambda i: (0, 0)),
            pl.BlockSpec((1, hdim), lambda i: (0, 0)),
            pl.BlockSpec(w2.shape, lambda i: (0, 0)),
            pl.BlockSpec((1, hdim), lambda i: (0, 0)),
        ],
        out_specs=pl.BlockSpec((bq, hdim), lambda i: (i, 0)),
        out_shape=jax.ShapeDtypeStruct((qn, hdim), jnp.float32),
    )(gath, pos_dst8, w1pad, w1p8, b1.reshape(1, hdim), w2,
      b2.reshape(1, hdim))


def _pad_cols(a, width):
    return jnp.pad(a, ((0, 0), (0, width - a.shape[1])))


def _stage(pos_src, pos_dst, x_src, w1, b1, w2, b2, r, ch):
    n = pos_src.shape[0]
    f = x_src.shape[1]
    hdim = w2.shape[0]
    p = 128
    table = _pad_cols(jnp.concatenate([x_src, pos_src], axis=1), p)
    pos_dst8 = _pad_cols(pos_dst, 8)
    pos_src8t = _pad_cols(pos_src, 8).T
    d2 = _d2(pos_dst8, pos_src8t)
    gath = _scan_gather(d2, table, jnp.float32(r * r), ch)
    w1pad = jnp.pad(w1, ((0, p - (f + 3)), (0, 0)))
    w1p8 = jnp.pad(w1[f:f + 3], ((0, 5), (0, 0)))
    return _conv(gath, pos_dst8, w1pad, w1p8, b1, w2, b2)


def kernel(pos, W11, b11, W12, b12, W21, b21, W22, b22, W31, b31, W32, b32, batch):
    n1 = N_POINTS // 2
    idx1 = _fps(pos, n1)
    pos1 = jnp.take(pos, idx1, axis=0)
    x1 = _stage(pos, pos1, pos, W11, b11, W12, b12, 0.2, 1024)

    n2 = n1 // 4
    idx2 = _fps(pos1, n2)
    pos2 = jnp.take(pos1, idx2, axis=0)
    x2 = _stage(pos1, pos2, x1, W21, b21, W22, b22, 0.4, 512)

    x3 = _stage(pos2, pos2, x2, W31, b31, W32, b32, 1.0, 512)

    batch3 = jnp.take(jnp.take(batch, idx1), idx2)
    return (x3, pos2, batch3)


# speedup vs baseline: 1.2710x; 1.0205x over previous
import functools

import jax
import jax.numpy as jnp
from jax.experimental import pallas as pl
from jax.experimental.pallas import tpu as pltpu
from jax.experimental.pallas import tpu_sc as plsc

N_POINTS = 8192
OUT_CHANNELS = 128
MAX_NBRS = 64

_NC = 2
_NS = 16
_NW = _NC * _NS
_BIG = 2 ** 30


def _fps_body(n_samples, n_points, xs_ref, ys_ref, zs_ref,
              xc_ref, yc_ref, zc_ref, sel_ref):
    xs = xs_ref[...]
    ys = ys_ref[...]
    zs = zs_ref[...]
    rows, lanes = xs.shape
    flat_iota = (jax.lax.broadcasted_iota(jnp.int32, (rows, lanes), 0) * lanes
                 + jax.lax.broadcasted_iota(jnp.int32, (rows, lanes), 1))

    srows = sel_ref.shape[0]
    sel_iota = (jax.lax.broadcasted_iota(jnp.int32, (srows, lanes), 0) * lanes
                + jax.lax.broadcasted_iota(jnp.int32, (srows, lanes), 1))

    def body(i, state):
        sel_vec, dists, lx, ly, lz = state
        dx = xs - lx
        dy = ys - ly
        dz = zs - lz
        d = dx * dx + dy * dy + dz * dz
        dists = jnp.minimum(dists, d)
        m = jnp.max(dists)
        idx = jnp.min(jnp.where(dists == m, flat_iota, n_points))
        sel_vec = jnp.where(sel_iota == i, idx, sel_vec)
        lx = xc_ref[idx, 0]
        ly = yc_ref[idx, 0]
        lz = zc_ref[idx, 0]
        return (sel_vec, dists, lx, ly, lz)

    sel0 = jnp.zeros((srows, lanes), dtype=jnp.int32)
    d0 = jnp.full((rows, lanes), jnp.inf, dtype=jnp.float32)
    sel, _, _, _, _ = jax.lax.fori_loop(
        1, n_samples, body,
        (sel0, d0, xc_ref[0, 0], yc_ref[0, 0], zc_ref[0, 0]))
    sel_ref[...] = sel


def _fps(pos, n_samples):
    n = pos.shape[0]
    xs = pos[:, 0].reshape(n // 128, 128)
    ys = pos[:, 1].reshape(n // 128, 128)
    zs = pos[:, 2].reshape(n // 128, 128)
    xc = pos[:, 0].reshape(n, 1)
    yc = pos[:, 1].reshape(n, 1)
    zc = pos[:, 2].reshape(n, 1)
    sel = pl.pallas_call(
        functools.partial(_fps_body, n_samples, n),
        out_shape=jax.ShapeDtypeStruct((n_samples // 128, 128), jnp.int32),
    )(xs, ys, zs, xc, yc, zc)
    return sel.reshape(n_samples)


def _d2_body(dst_ref, srcT_ref, out_ref):
    q = dst_ref[...]
    sT = srcT_ref[...]
    qs2 = jnp.sum(q * q, axis=1, keepdims=True)
    ss2 = jnp.sum(sT * sT, axis=0, keepdims=True)
    cross = jnp.dot(q, sT, preferred_element_type=jnp.float32)
    out_ref[...] = qs2 + ss2 - 2.0 * cross


def _d2(pos_dst8, pos_src8T):
    qn = pos_dst8.shape[0]
    n = pos_src8T.shape[1]
    bd = 256
    return pl.pallas_call(
        _d2_body,
        grid=(qn // bd,),
        in_specs=[
            pl.BlockSpec((bd, 8), lambda i: (i, 0)),
            pl.BlockSpec((8, n), lambda i: (0, 0)),
        ],
        out_specs=pl.BlockSpec((bd, n), lambda i: (i, 0)),
        out_shape=jax.ShapeDtypeStruct((qn, n), jnp.float32),
    )(pos_dst8, pos_src8T)


def _scan_body(rows_pw, n, ch, r2, d2_hbm, tab_hbm, gath_hbm,
               chunk_v, idx_v, gath_v, sem, wsem):
    cid = jax.lax.axis_index("c")
    sid = jax.lax.axis_index("s")
    wid = sid * _NC + cid
    iota16 = jax.lax.iota(jnp.int32, 16)
    nchunks = n // ch
    nvr = ch // 16

    def row_body(rl, carry):
        q = wid * rows_pw + rl
        pltpu.async_copy(d2_hbm.at[q, pl.ds(0, ch)], chunk_v.at[0], sem)

        def cond(st):
            i, cnt = st
            return jnp.logical_and(i < nchunks, cnt < MAX_NBRS)

        def cbody(st):
            i, cnt = st
            b = jax.lax.rem(i, 2)
            pltpu.make_async_copy(
                d2_hbm.at[q, pl.ds(0, ch)], chunk_v.at[b], sem).wait()

            @pl.when(i + 1 < nchunks)
            def _():
                pltpu.async_copy(
                    d2_hbm.at[q, pl.ds((i + 1) * ch, ch)],
                    chunk_v.at[jax.lax.rem(i + 1, 2)], sem)

            base = i * ch

            def vbody(v, cnt2):
                d = chunk_v[b, pl.ds(v * 16, 16)]
                m = d <= r2
                nv = plsc.all_reduce_population_count(m)[0]

                @pl.when(nv > 0)
                def _():
                    csum = plsc.cumsum(m.astype(jnp.int32))
                    slot = cnt2 + csum - 1
                    glob = iota16 + (base + v * 16)
                    msel = jnp.logical_and(m, slot < MAX_NBRS)
                    plsc.store_scatter(idx_v, [slot], glob, mask=msel)

                return cnt2 + nv

            cnt = jax.lax.fori_loop(0, nvr, vbody, cnt)
            return (i + 1, cnt)

        i_exit, cnt = jax.lax.while_loop(
            cond, cbody, (jnp.int32(0), jnp.int32(0)))

        @pl.when(i_exit < nchunks)
        def _():
            pltpu.make_async_copy(
                d2_hbm.at[q, pl.ds(0, ch)], chunk_v.at[0], sem).wait()

        first = idx_v[pl.ds(0, 16)][0]
        for k in range(MAX_NBRS // 16):
            sp = iota16 + k * 16
            cur = idx_v[pl.ds(k * 16, 16)]
            idx_v[pl.ds(k * 16, 16)] = jnp.where(sp < cnt, cur, first)

        gb = jax.lax.rem(rl, 2)

        @pl.when(rl >= 2)
        def _():
            pltpu.make_async_copy(
                gath_v.at[gb], gath_hbm.at[pl.ds(0, MAX_NBRS)], wsem).wait()

        pltpu.async_copy(tab_hbm.at[idx_v], gath_v.at[gb], sem).wait()
        pltpu.async_copy(
            gath_v.at[gb], gath_hbm.at[pl.ds(q * MAX_NBRS, MAX_NBRS)], wsem)
        return carry

    jax.lax.fori_loop(0, rows_pw, row_body, jnp.int32(0))
    for _ in range(2):
        pltpu.make_async_copy(
            gath_v.at[0], gath_hbm.at[pl.ds(0, MAX_NBRS)], wsem).wait()


def _scan_gather(d2, table, r2, ch):
    qn, n = d2.shape
    p = table.shape[1]
    rows_pw = qn // _NW
    mesh = plsc.VectorSubcoreMesh(
        core_axis_name="c", subcore_axis_name="s",
        num_cores=_NC, num_subcores=_NS)
    return pl.kernel(
        functools.partial(_scan_body, rows_pw, n, ch, r2),
        out_type=jax.ShapeDtypeStruct((qn * MAX_NBRS, p), jnp.float32),
        mesh=mesh,
        scratch_types=[
            pltpu.VMEM((2, ch), jnp.float32),
            pltpu.VMEM((MAX_NBRS,), jnp.int32),
            pltpu.VMEM((2, MAX_NBRS, p), jnp.float32),
            pltpu.SemaphoreType.DMA,
            pltpu.SemaphoreType.DMA,
        ],
        compiler_params=pltpu.CompilerParams(needs_layout_passes=False),
    )(d2, table)


def _conv_body(bq, hdim, gath_ref, posd_ref, w1_ref, w1p_ref, b1_ref,
               w2_ref, b2_ref, out_ref):
    g = gath_ref[...]
    e = jnp.dot(g, w1_ref[...],
                preferred_element_type=jnp.float32) + b1_ref[...]
    wq = jnp.dot(posd_ref[...], w1p_ref[...],
                 preferred_element_type=jnp.float32)
    e3 = e.reshape(bq, MAX_NBRS, hdim)
    h3 = jnp.maximum(e3 - wq[:, None, :], 0.0)
    msg = jnp.dot(h3.reshape(bq * MAX_NBRS, hdim), w2_ref[...],
                  preferred_element_type=jnp.float32)
    m3 = msg.reshape(bq, MAX_NBRS, hdim)
    out_ref[...] = jnp.max(m3, axis=1) + b2_ref[...]


def _conv(gath, pos_dst8, w1pad, w1p8, b1, w2, b2):
    qn = pos_dst8.shape[0]
    p = gath.shape[1]
    hdim = w2.shape[0]
    bq = 128
    return pl.pallas_call(
        functools.partial(_conv_body, bq, hdim),
        grid=(qn // bq,),
        in_specs=[
            pl.BlockSpec((bq * MAX_NBRS, p), lambda i: (i, 0)),
            pl.BlockSpec((bq, 8), lambda i: (i, 0)),
            pl.BlockSpec(w1pad.shape, lambda i: (0, 0)),
            pl.BlockSpec(w1p8.shape, lambda i: (0, 0)),
            pl.BlockSpec((1, hdim), lambda i: (0, 0)),
            pl.BlockSpec(w2.shape, lambda i: (0, 0)),
            pl.BlockSpec((1, hdim), lambda i: (0, 0)),
        ],
        out_specs=pl.BlockSpec((bq, hdim), lambda i: (i, 0)),
        out_shape=jax.ShapeDtypeStruct((qn, hdim), jnp.float32),
    )(gath, pos_dst8, w1pad, w1p8, b1.reshape(1, hdim), w2,
      b2.reshape(1, hdim))


def _pad_cols(a, width):
    return jnp.pad(a, ((0, 0), (0, width - a.shape[1])))


def _stage(pos_src, pos_dst, x_src, w1, b1, w2, b2, r, ch):
    n = pos_src.shape[0]
    f = x_src.shape[1]
    hdim = w2.shape[0]
    p = 128
    table = _pad_cols(jnp.concatenate([x_src, pos_src], axis=1), p)
    pos_dst8 = _pad_cols(pos_dst, 8)
    pos_src8t = _pad_cols(pos_src, 8).T
    d2 = _d2(pos_dst8, pos_src8t)
    gath = _scan_gather(d2, table, jnp.float32(r * r), ch)
    w1pad = jnp.pad(w1, ((0, p - (f + 3)), (0, 0)))
    w1p8 = jnp.pad(w1[f:f + 3], ((0, 5), (0, 0)))
    return _conv(gath, pos_dst8, w1pad, w1p8, b1, w2, b2)


def kernel(pos, W11, b11, W12, b12, W21, b21, W22, b22, W31, b31, W32, b32, batch):
    n1 = N_POINTS // 2
    idx1 = _fps(pos, n1)
    pos1 = jnp.take(pos, idx1, axis=0)
    x1 = _stage(pos, pos1, pos, W11, b11, W12, b12, 0.2, 1024)

    n2 = n1 // 4
    idx2 = _fps(pos1, n2)
    pos2 = jnp.take(pos1, idx2, axis=0)
    x2 = _stage(pos1, pos2, x1, W21, b21, W22, b22, 0.4, 512)

    x3 = _stage(pos2, pos2, x2, W31, b31, W32, b32, 1.0, 512)

    batch3 = jnp.take(jnp.take(batch, idx1), idx2)
    return (x3, pos2, batch3)
